# initial kernel scaffold (unmeasured)
import jax
import jax.numpy as jnp
from jax import lax
from jax.experimental import pallas as pl
from jax.experimental.pallas import tpu as pltpu

N_DEV = 16
SQ = 1024
D_MODEL = 1024
HQ_PER = 8
DH = 128
HEAD_COLS = HQ_PER * DH
CHUNK = SQ // N_DEV
SCALE = 0.08838834764831843
BLK = 64


def _body(x_ref, wq_ref, k_ref, v_ref, wo_ref, out_ref,
          q_ref, ctx_ref, acc_ref, rs_recv_ref,
          send_sem, rs_recv_sems, ag_recv_sems):
    my = lax.axis_index("i")
    left = (my - 1) % N_DEV
    right = (my + 1) % N_DEV

    barrier_sem = pltpu.get_barrier_semaphore()
    pl.semaphore_signal(barrier_sem, inc=1, device_id=(left,),
                        device_id_type=pl.DeviceIdType.MESH)
    pl.semaphore_signal(barrier_sem, inc=1, device_id=(right,),
                        device_id_type=pl.DeviceIdType.MESH)
    pl.semaphore_wait(barrier_sem, 2)

    q_ref[...] = jnp.dot(x_ref[...], wq_ref[...],
                         preferred_element_type=jnp.float32)

    qb = lax.broadcasted_iota(jnp.int32, (SQ, SQ), 0) // BLK
    kb = lax.broadcasted_iota(jnp.int32, (SQ, SQ), 1) // BLK
    mask = kb <= qb

    for h in range(HQ_PER):
        sl = slice(h * DH, (h + 1) * DH)
        qh = q_ref[:, sl]
        kh = k_ref[:, sl]
        s = lax.dot_general(qh, kh, (((1,), (1,)), ((), ())),
                            preferred_element_type=jnp.float32) * SCALE
        s = jnp.where(mask, s, -1e9)
        m = jnp.max(s, axis=1, keepdims=True)
        w = jnp.exp(s - m)
        w = w / jnp.sum(w, axis=1, keepdims=True)
        ctx_ref[:, sl] = jnp.dot(w, v_ref[:, sl],
                                 preferred_element_type=jnp.float32)

    acc_ref[...] = jnp.dot(ctx_ref[...], wo_ref[...],
                           preferred_element_type=jnp.float32)

    for s in range(N_DEV - 1):
        c_send = (my - s) % N_DEV
        c_recv = (my - s - 1) % N_DEV
        rdma = pltpu.make_async_remote_copy(
            src_ref=acc_ref.at[pl.ds(c_send * CHUNK, CHUNK), :],
            dst_ref=rs_recv_ref.at[s],
            send_sem=send_sem,
            recv_sem=rs_recv_sems.at[s],
            device_id=(right,),
            device_id_type=pl.DeviceIdType.MESH,
        )
        rdma.start()
        rdma.wait()
        cur = pl.load(acc_ref, (pl.ds(c_recv * CHUNK, CHUNK), slice(None)))
        pl.store(acc_ref, (pl.ds(c_recv * CHUNK, CHUNK), slice(None)),
                 cur + rs_recv_ref[s])

    c_mine = (my + 1) % N_DEV
    pl.store(out_ref, (pl.ds(c_mine * CHUNK, CHUNK), slice(None)),
             pl.load(acc_ref, (pl.ds(c_mine * CHUNK, CHUNK), slice(None))))

    for s in range(N_DEV - 1):
        c_send = (my + 1 - s) % N_DEV
        rdma = pltpu.make_async_remote_copy(
            src_ref=out_ref.at[pl.ds(c_send * CHUNK, CHUNK), :],
            dst_ref=out_ref.at[pl.ds(c_send * CHUNK, CHUNK), :],
            send_sem=send_sem,
            recv_sem=ag_recv_sems.at[s],
            device_id=(right,),
            device_id_type=pl.DeviceIdType.MESH,
        )
        rdma.start()
        rdma.wait()


def kernel(x, Wq, K_ext, V_ext, Wo):
    my = lax.axis_index("i")
    x2 = x.reshape(SQ, D_MODEL)
    k2 = K_ext.reshape(SQ, HEAD_COLS)
    v2 = V_ext.reshape(SQ, HEAD_COLS)
    wq_s = lax.dynamic_slice(Wq, (0, my * HEAD_COLS), (D_MODEL, HEAD_COLS))
    wo_s = lax.dynamic_slice(Wo, (my * HEAD_COLS, 0), (HEAD_COLS, D_MODEL))

    out = pl.pallas_call(
        _body,
        out_shape=jax.ShapeDtypeStruct((SQ, D_MODEL), jnp.float32),
        in_specs=[pl.BlockSpec(memory_space=pltpu.VMEM)] * 5,
        out_specs=pl.BlockSpec(memory_space=pltpu.VMEM),
        scratch_shapes=[
            pltpu.VMEM((SQ, HEAD_COLS), jnp.float32),
            pltpu.VMEM((SQ, HEAD_COLS), jnp.float32),
            pltpu.VMEM((SQ, D_MODEL), jnp.float32),
            pltpu.VMEM((N_DEV - 1, CHUNK, D_MODEL), jnp.float32),
            pltpu.SemaphoreType.DMA,
            pltpu.SemaphoreType.DMA((N_DEV - 1,)),
            pltpu.SemaphoreType.DMA((N_DEV - 1,)),
        ],
        compiler_params=pltpu.CompilerParams(collective_id=0),
    )(x2, wq_s, k2, v2, wo_s)
    return out.reshape(1, SQ, D_MODEL)


# baseline (device time: 176837 ns/iter reference)
import jax
import jax.numpy as jnp
from jax import lax
from jax.experimental import pallas as pl
from jax.experimental.pallas import tpu as pltpu

N_DEV = 16
SQ = 1024
D_MODEL = 1024
HQ_PER = 8
DH = 128
HEAD_COLS = HQ_PER * DH
CHUNK = SQ // N_DEV
SCALE = 0.08838834764831843
BLK = 64


def _body(x_ref, wq_ref, k_ref, v_ref, wo_ref, out_ref,
          q_ref, ctx_ref, acc_ref, rs_recv_ref,
          send_sem, rs_recv_sems, ag_recv_sems):
    my = lax.axis_index("i")
    left = (my - 1) % N_DEV
    right = (my + 1) % N_DEV

    barrier_sem = pltpu.get_barrier_semaphore()
    pl.semaphore_signal(barrier_sem, inc=1, device_id=(left,),
                        device_id_type=pl.DeviceIdType.MESH)
    pl.semaphore_signal(barrier_sem, inc=1, device_id=(right,),
                        device_id_type=pl.DeviceIdType.MESH)
    pl.semaphore_wait(barrier_sem, 2)

    q_ref[...] = jnp.dot(x_ref[...], wq_ref[...],
                         preferred_element_type=jnp.float32)

    qb = lax.broadcasted_iota(jnp.int32, (SQ, SQ), 0) // BLK
    kb = lax.broadcasted_iota(jnp.int32, (SQ, SQ), 1) // BLK
    mask = kb <= qb

    for h in range(HQ_PER):
        sl = slice(h * DH, (h + 1) * DH)
        qh = q_ref[:, sl]
        kh = k_ref[:, sl]
        s = lax.dot_general(qh, kh, (((1,), (1,)), ((), ())),
                            preferred_element_type=jnp.float32) * SCALE
        s = jnp.where(mask, s, -1e9)
        m = jnp.max(s, axis=1, keepdims=True)
        w = jnp.exp(s - m)
        w = w / jnp.sum(w, axis=1, keepdims=True)
        ctx_ref[:, sl] = jnp.dot(w, v_ref[:, sl],
                                 preferred_element_type=jnp.float32)

    acc_ref[...] = jnp.dot(ctx_ref[...], wo_ref[...],
                           preferred_element_type=jnp.float32)

    for s in range(N_DEV - 1):
        c_send = (my - s) % N_DEV
        c_recv = (my - s - 1) % N_DEV
        rdma = pltpu.make_async_remote_copy(
            src_ref=acc_ref.at[pl.ds(c_send * CHUNK, CHUNK), :],
            dst_ref=rs_recv_ref.at[s],
            send_sem=send_sem,
            recv_sem=rs_recv_sems.at[s],
            device_id=(right,),
            device_id_type=pl.DeviceIdType.MESH,
        )
        rdma.start()
        rdma.wait()
        acc_ref[pl.ds(c_recv * CHUNK, CHUNK), :] = (
            acc_ref[pl.ds(c_recv * CHUNK, CHUNK), :] + rs_recv_ref[s]
        )

    c_mine = (my + 1) % N_DEV
    out_ref[pl.ds(c_mine * CHUNK, CHUNK), :] = (
        acc_ref[pl.ds(c_mine * CHUNK, CHUNK), :]
    )

    for s in range(N_DEV - 1):
        c_send = (my + 1 - s) % N_DEV
        rdma = pltpu.make_async_remote_copy(
            src_ref=out_ref.at[pl.ds(c_send * CHUNK, CHUNK), :],
            dst_ref=out_ref.at[pl.ds(c_send * CHUNK, CHUNK), :],
            send_sem=send_sem,
            recv_sem=ag_recv_sems.at[s],
            device_id=(right,),
            device_id_type=pl.DeviceIdType.MESH,
        )
        rdma.start()
        rdma.wait()


def kernel(x, Wq, K_ext, V_ext, Wo):
    my = lax.axis_index("i")
    x2 = x.reshape(SQ, D_MODEL)
    k2 = K_ext.reshape(SQ, HEAD_COLS)
    v2 = V_ext.reshape(SQ, HEAD_COLS)
    wq_s = lax.dynamic_slice(Wq, (0, my * HEAD_COLS), (D_MODEL, HEAD_COLS))
    wo_s = lax.dynamic_slice(Wo, (my * HEAD_COLS, 0), (HEAD_COLS, D_MODEL))

    out = pl.pallas_call(
        _body,
        out_shape=jax.ShapeDtypeStruct((SQ, D_MODEL), jnp.float32),
        in_specs=[pl.BlockSpec(memory_space=pltpu.VMEM)] * 5,
        out_specs=pl.BlockSpec(memory_space=pltpu.VMEM),
        scratch_shapes=[
            pltpu.VMEM((SQ, HEAD_COLS), jnp.float32),
            pltpu.VMEM((SQ, HEAD_COLS), jnp.float32),
            pltpu.VMEM((SQ, D_MODEL), jnp.float32),
            pltpu.VMEM((N_DEV - 1, CHUNK, D_MODEL), jnp.float32),
            pltpu.SemaphoreType.DMA,
            pltpu.SemaphoreType.DMA((N_DEV - 1,)),
            pltpu.SemaphoreType.DMA((N_DEV - 1,)),
        ],
        compiler_params=pltpu.CompilerParams(collective_id=0),
    )(x2, wq_s, k2, v2, wo_s)
    return out.reshape(1, SQ, D_MODEL)


# device time: 96415 ns/iter; 1.8341x vs baseline; 1.8341x over previous
import jax
import jax.numpy as jnp
from jax import lax
from jax.experimental import pallas as pl
from jax.experimental.pallas import tpu as pltpu

N_DEV = 16
SQ = 1024
D_MODEL = 1024
HQ_PER = 8
DH = 128
HEAD_COLS = HQ_PER * DH
CHUNK = SQ // N_DEV
SCALE = 0.08838834764831843
BLK = 64

MASKS = (1, 3, 4, 8)
RS_SIZES = (512, 256, 128, 64)
RS_OFF = (0, 512, 768, 896)
PERM = tuple(b ^ 1 if b % 4 >= 2 else b for b in range(N_DEV))


def _body(x_ref, wq_ref, k_ref, v_ref, wo_ref, out_ref,
          q_ref, ctx_ref, acc_ref, send_ref, rs_recv_ref, ag_ref,
          send_sem, rs_sems, ag_sems):
    my = lax.axis_index("i")

    barrier_sem = pltpu.get_barrier_semaphore()
    for msk in MASKS:
        pl.semaphore_signal(barrier_sem, inc=1, device_id=(my ^ msk,),
                            device_id_type=pl.DeviceIdType.MESH)
    pl.semaphore_wait(barrier_sem, len(MASKS))

    q_ref[...] = jnp.dot(x_ref[...], wq_ref[...],
                         preferred_element_type=jnp.float32)

    qb = lax.broadcasted_iota(jnp.int32, (SQ, SQ), 0) // BLK
    kb = lax.broadcasted_iota(jnp.int32, (SQ, SQ), 1) // BLK
    mask = kb <= qb

    for h in range(HQ_PER):
        sl = slice(h * DH, (h + 1) * DH)
        qh = q_ref[:, sl]
        kh = k_ref[:, sl]
        s = lax.dot_general(qh, kh, (((1,), (1,)), ((), ())),
                            preferred_element_type=jnp.float32) * SCALE
        s = jnp.where(mask, s, -1e9)
        m = jnp.max(s, axis=1, keepdims=True)
        w = jnp.exp(s - m)
        w = w / jnp.sum(w, axis=1, keepdims=True)
        ctx_ref[:, sl] = jnp.dot(w, v_ref[:, sl],
                                 preferred_element_type=jnp.float32)

    val = jnp.dot(ctx_ref[...], wo_ref[...],
                  preferred_element_type=jnp.float32)
    for q in range(N_DEV):
        pq = PERM[q]
        acc_ref[q * CHUNK:(q + 1) * CHUNK, :] = (
            val[pq * CHUNK:(pq + 1) * CHUNK, :]
        )

    bit0 = my & 1
    bit1 = (my >> 1) & 1
    betas = (bit0 ^ bit1, bit1, (my >> 2) & 1, (my >> 3) & 1)

    lo = my * 0
    for k in range(4):
        s = RS_SIZES[k]
        partner = my ^ MASKS[k]
        send_lo = lo + (1 - betas[k]) * s
        keep_lo = lo + betas[k] * s
        send_ref[0:s, :] = acc_ref[pl.ds(pl.multiple_of(send_lo, 64), s), :].astype(jnp.bfloat16)
        rdma = pltpu.make_async_remote_copy(
            src_ref=send_ref.at[0:s],
            dst_ref=rs_recv_ref.at[RS_OFF[k]:RS_OFF[k] + s],
            send_sem=send_sem,
            recv_sem=rs_sems.at[k],
            device_id=(partner,),
            device_id_type=pl.DeviceIdType.MESH,
        )
        rdma.start()
        rdma.wait()
        keep_lo = pl.multiple_of(keep_lo, 64)
        acc_ref[pl.ds(keep_lo, s), :] = (
            acc_ref[pl.ds(keep_lo, s), :]
            + rs_recv_ref[RS_OFF[k]:RS_OFF[k] + s, :].astype(jnp.float32)
        )
        lo = keep_lo

    ag_ref[pl.ds(pl.multiple_of(lo, 64), CHUNK), :] = acc_ref[pl.ds(pl.multiple_of(lo, 64), CHUNK), :].astype(
        jnp.bfloat16)

    for k in reversed(range(4)):
        sz = CHUNK << (3 - k)
        partner = my ^ MASKS[k]
        rdma = pltpu.make_async_remote_copy(
            src_ref=ag_ref.at[pl.ds(pl.multiple_of(lo, 64), sz)],
            dst_ref=ag_ref.at[pl.ds(pl.multiple_of(lo, 64), sz)],
            send_sem=send_sem,
            recv_sem=ag_sems.at[k],
            device_id=(partner,),
            device_id_type=pl.DeviceIdType.MESH,
        )
        rdma.start()
        rdma.wait()
        lo = lo - (lo & sz)

    for b in range(N_DEV):
        pb = PERM[b]
        out_ref[b * CHUNK:(b + 1) * CHUNK, :] = (
            ag_ref[pb * CHUNK:(pb + 1) * CHUNK, :].astype(jnp.float32)
        )


def kernel(x, Wq, K_ext, V_ext, Wo):
    my = lax.axis_index("i")
    x2 = x.reshape(SQ, D_MODEL)
    k2 = K_ext.reshape(SQ, HEAD_COLS)
    v2 = V_ext.reshape(SQ, HEAD_COLS)
    wq_s = lax.dynamic_slice(Wq, (0, my * HEAD_COLS), (D_MODEL, HEAD_COLS))
    wo_s = lax.dynamic_slice(Wo, (my * HEAD_COLS, 0), (HEAD_COLS, D_MODEL))

    out = pl.pallas_call(
        _body,
        out_shape=jax.ShapeDtypeStruct((SQ, D_MODEL), jnp.float32),
        in_specs=[pl.BlockSpec(memory_space=pltpu.VMEM)] * 5,
        out_specs=pl.BlockSpec(memory_space=pltpu.VMEM),
        scratch_shapes=[
            pltpu.VMEM((SQ, HEAD_COLS), jnp.float32),
            pltpu.VMEM((SQ, HEAD_COLS), jnp.float32),
            pltpu.VMEM((SQ, D_MODEL), jnp.float32),
            pltpu.VMEM((512, D_MODEL), jnp.bfloat16),
            pltpu.VMEM((960, D_MODEL), jnp.bfloat16),
            pltpu.VMEM((SQ, D_MODEL), jnp.bfloat16),
            pltpu.SemaphoreType.DMA,
            pltpu.SemaphoreType.DMA((4,)),
            pltpu.SemaphoreType.DMA((4,)),
        ],
        compiler_params=pltpu.CompilerParams(collective_id=0),
    )(x2, wq_s, k2, v2, wo_s)
    return out.reshape(1, SQ, D_MODEL)


# device time: 93462 ns/iter; 1.8921x vs baseline; 1.0316x over previous
import jax
import jax.numpy as jnp
from jax import lax
from jax.experimental import pallas as pl
from jax.experimental.pallas import tpu as pltpu

N_DEV = 16
SQ = 1024
D_MODEL = 1024
HQ_PER = 8
DH = 128
HEAD_COLS = HQ_PER * DH
CHUNK = SQ // N_DEV
SCALE = 0.08838834764831843
BLK = 64

MASKS = (1, 3, 4, 8)
RS_SIZES = (512, 256, 128, 64)
RS_OFF = (0, 512, 768, 896)
PERM = tuple(b ^ 1 if b % 4 >= 2 else b for b in range(N_DEV))


def _body(x_ref, wq_ref, k_ref, v_ref, wo_ref, out_ref,
          q_ref, ctx_ref, acc_ref, send_ref, rs_recv_ref, ag_ref,
          send_sem, rs_sems, ag_sems):
    my = lax.axis_index("i")

    barrier_sem = pltpu.get_barrier_semaphore()
    for msk in MASKS:
        pl.semaphore_signal(barrier_sem, inc=1, device_id=(my ^ msk,),
                            device_id_type=pl.DeviceIdType.MESH)
    pl.semaphore_wait(barrier_sem, len(MASKS))

    q_ref[...] = jnp.dot(x_ref[...], wq_ref[...],
                         preferred_element_type=jnp.float32).astype(jnp.bfloat16)

    qb = lax.broadcasted_iota(jnp.int32, (SQ, SQ), 0) // BLK
    kb = lax.broadcasted_iota(jnp.int32, (SQ, SQ), 1) // BLK
    mask = kb <= qb

    for h in range(HQ_PER):
        sl = slice(h * DH, (h + 1) * DH)
        qh = q_ref[:, sl]
        kh = k_ref[:, sl]
        s = lax.dot_general(qh, kh, (((1,), (1,)), ((), ())),
                            preferred_element_type=jnp.float32) * SCALE
        s = jnp.where(mask, s, -1e9)
        m = jnp.max(s, axis=1, keepdims=True)
        w = jnp.exp(s - m)
        w = (w / jnp.sum(w, axis=1, keepdims=True)).astype(jnp.bfloat16)
        ctx_ref[:, sl] = jnp.dot(w, v_ref[:, sl],
                                 preferred_element_type=jnp.float32
                                 ).astype(jnp.bfloat16)

    val = jnp.dot(ctx_ref[...], wo_ref[...],
                  preferred_element_type=jnp.float32)
    for q in range(N_DEV):
        pq = PERM[q]
        acc_ref[q * CHUNK:(q + 1) * CHUNK, :] = (
            val[pq * CHUNK:(pq + 1) * CHUNK, :]
        )

    bit0 = my & 1
    bit1 = (my >> 1) & 1
    betas = (bit0 ^ bit1, bit1, (my >> 2) & 1, (my >> 3) & 1)

    lo = my * 0
    for k in range(4):
        s = RS_SIZES[k]
        partner = my ^ MASKS[k]
        send_lo = lo + (1 - betas[k]) * s
        keep_lo = lo + betas[k] * s
        send_ref[0:s, :] = acc_ref[pl.ds(pl.multiple_of(send_lo, 64), s), :].astype(jnp.bfloat16)
        rdma = pltpu.make_async_remote_copy(
            src_ref=send_ref.at[0:s],
            dst_ref=rs_recv_ref.at[RS_OFF[k]:RS_OFF[k] + s],
            send_sem=send_sem,
            recv_sem=rs_sems.at[k],
            device_id=(partner,),
            device_id_type=pl.DeviceIdType.MESH,
        )
        rdma.start()
        rdma.wait()
        keep_lo = pl.multiple_of(keep_lo, 64)
        acc_ref[pl.ds(keep_lo, s), :] = (
            acc_ref[pl.ds(keep_lo, s), :]
            + rs_recv_ref[RS_OFF[k]:RS_OFF[k] + s, :].astype(jnp.float32)
        )
        lo = keep_lo

    ag_ref[pl.ds(pl.multiple_of(lo, 64), CHUNK), :] = acc_ref[pl.ds(pl.multiple_of(lo, 64), CHUNK), :].astype(
        jnp.bfloat16)

    for k in reversed(range(4)):
        sz = CHUNK << (3 - k)
        partner = my ^ MASKS[k]
        rdma = pltpu.make_async_remote_copy(
            src_ref=ag_ref.at[pl.ds(pl.multiple_of(lo, 64), sz)],
            dst_ref=ag_ref.at[pl.ds(pl.multiple_of(lo, 64), sz)],
            send_sem=send_sem,
            recv_sem=ag_sems.at[k],
            device_id=(partner,),
            device_id_type=pl.DeviceIdType.MESH,
        )
        rdma.start()
        rdma.wait()
        lo = lo - (lo & sz)

    for b in range(N_DEV):
        pb = PERM[b]
        out_ref[b * CHUNK:(b + 1) * CHUNK, :] = (
            ag_ref[pb * CHUNK:(pb + 1) * CHUNK, :].astype(jnp.float32)
        )


def kernel(x, Wq, K_ext, V_ext, Wo):
    my = lax.axis_index("i")
    x2 = x.reshape(SQ, D_MODEL).astype(jnp.bfloat16)
    k2 = K_ext.reshape(SQ, HEAD_COLS).astype(jnp.bfloat16)
    v2 = V_ext.reshape(SQ, HEAD_COLS).astype(jnp.bfloat16)
    wq_s = lax.dynamic_slice(Wq, (0, my * HEAD_COLS), (D_MODEL, HEAD_COLS)).astype(jnp.bfloat16)
    wo_s = lax.dynamic_slice(Wo, (my * HEAD_COLS, 0), (HEAD_COLS, D_MODEL)).astype(jnp.bfloat16)

    out = pl.pallas_call(
        _body,
        out_shape=jax.ShapeDtypeStruct((SQ, D_MODEL), jnp.float32),
        in_specs=[pl.BlockSpec(memory_space=pltpu.VMEM)] * 5,
        out_specs=pl.BlockSpec(memory_space=pltpu.VMEM),
        scratch_shapes=[
            pltpu.VMEM((SQ, HEAD_COLS), jnp.bfloat16),
            pltpu.VMEM((SQ, HEAD_COLS), jnp.bfloat16),
            pltpu.VMEM((SQ, D_MODEL), jnp.float32),
            pltpu.VMEM((512, D_MODEL), jnp.bfloat16),
            pltpu.VMEM((960, D_MODEL), jnp.bfloat16),
            pltpu.VMEM((SQ, D_MODEL), jnp.bfloat16),
            pltpu.SemaphoreType.DMA,
            pltpu.SemaphoreType.DMA((4,)),
            pltpu.SemaphoreType.DMA((4,)),
        ],
        compiler_params=pltpu.CompilerParams(collective_id=0),
    )(x2, wq_s, k2, v2, wo_s)
    return out.reshape(1, SQ, D_MODEL)


# device time: 90060 ns/iter; 1.9635x vs baseline; 1.0378x over previous
import jax
import jax.numpy as jnp
from jax import lax
from jax.experimental import pallas as pl
from jax.experimental.pallas import tpu as pltpu

N_DEV = 16
SQ = 1024
D_MODEL = 1024
HQ_PER = 8
DH = 128
HEAD_COLS = HQ_PER * DH
CHUNK = SQ // N_DEV
SCALE = 0.08838834764831843
BLK = 64

MASKS = (1, 3, 4, 8)
RS_SIZES = (512, 256, 128, 64)
RS_OFF = (0, 512, 768, 896)
PERM = tuple(b ^ 1 if b % 4 >= 2 else b for b in range(N_DEV))


def _body(x_ref, wq_ref, k_ref, v_ref, wo_ref, out_ref,
          q_ref, ctx_ref, acc_ref, send_ref, rs_recv_ref, ag_ref,
          bias_ref, send_sem, rs_sems, ag_sems):
    my = lax.axis_index("i")

    barrier_sem = pltpu.get_barrier_semaphore()
    for msk in MASKS:
        pl.semaphore_signal(barrier_sem, inc=1, device_id=(my ^ msk,),
                            device_id_type=pl.DeviceIdType.MESH)
    pl.semaphore_wait(barrier_sem, len(MASKS))

    q_ref[...] = jnp.dot(x_ref[...], wq_ref[...],
                         preferred_element_type=jnp.float32).astype(jnp.bfloat16)

    qb = lax.broadcasted_iota(jnp.int32, (SQ, SQ), 0) // BLK
    kb = lax.broadcasted_iota(jnp.int32, (SQ, SQ), 1) // BLK
    bias_ref[...] = jnp.where(kb <= qb, 0.0, -30000.0).astype(jnp.float32)

    HALF = SQ // 2
    for h in range(HQ_PER):
        sl = slice(h * DH, (h + 1) * DH)
        for (r0, r1, ncols) in ((0, HALF, HALF), (HALF, SQ, SQ)):
            qh = q_ref[r0:r1, sl]
            kh = k_ref[0:ncols, sl]
            s = lax.dot_general(qh, kh, (((1,), (1,)), ((), ())),
                                preferred_element_type=jnp.float32) * SCALE
            s = s + bias_ref[r0:r1, 0:ncols]
            w = jnp.exp(s)
            w = (w / jnp.sum(w, axis=1, keepdims=True)).astype(jnp.bfloat16)
            ctx_ref[r0:r1, sl] = jnp.dot(
                w, v_ref[0:ncols, sl],
                preferred_element_type=jnp.float32).astype(jnp.bfloat16)

    val = jnp.dot(ctx_ref[...], wo_ref[...],
                  preferred_element_type=jnp.float32)
    for q in range(N_DEV):
        pq = PERM[q]
        acc_ref[q * CHUNK:(q + 1) * CHUNK, :] = (
            val[pq * CHUNK:(pq + 1) * CHUNK, :]
        )

    bit0 = my & 1
    bit1 = (my >> 1) & 1
    betas = (bit0 ^ bit1, bit1, (my >> 2) & 1, (my >> 3) & 1)

    lo = my * 0
    for k in range(4):
        s = RS_SIZES[k]
        partner = my ^ MASKS[k]
        send_lo = lo + (1 - betas[k]) * s
        keep_lo = lo + betas[k] * s
        send_ref[0:s, :] = acc_ref[pl.ds(pl.multiple_of(send_lo, 64), s), :].astype(jnp.bfloat16)
        rdma = pltpu.make_async_remote_copy(
            src_ref=send_ref.at[0:s],
            dst_ref=rs_recv_ref.at[RS_OFF[k]:RS_OFF[k] + s],
            send_sem=send_sem,
            recv_sem=rs_sems.at[k],
            device_id=(partner,),
            device_id_type=pl.DeviceIdType.MESH,
        )
        rdma.start()
        rdma.wait()
        keep_lo = pl.multiple_of(keep_lo, 64)
        acc_ref[pl.ds(keep_lo, s), :] = (
            acc_ref[pl.ds(keep_lo, s), :]
            + rs_recv_ref[RS_OFF[k]:RS_OFF[k] + s, :].astype(jnp.float32)
        )
        lo = keep_lo

    ag_ref[pl.ds(pl.multiple_of(lo, 64), CHUNK), :] = acc_ref[pl.ds(pl.multiple_of(lo, 64), CHUNK), :].astype(
        jnp.bfloat16)

    for k in reversed(range(4)):
        sz = CHUNK << (3 - k)
        partner = my ^ MASKS[k]
        rdma = pltpu.make_async_remote_copy(
            src_ref=ag_ref.at[pl.ds(pl.multiple_of(lo, 64), sz)],
            dst_ref=ag_ref.at[pl.ds(pl.multiple_of(lo, 64), sz)],
            send_sem=send_sem,
            recv_sem=ag_sems.at[k],
            device_id=(partner,),
            device_id_type=pl.DeviceIdType.MESH,
        )
        rdma.start()
        rdma.wait()
        lo = lo - (lo & sz)

    for b in range(N_DEV):
        pb = PERM[b]
        out_ref[b * CHUNK:(b + 1) * CHUNK, :] = (
            ag_ref[pb * CHUNK:(pb + 1) * CHUNK, :].astype(jnp.float32)
        )


def kernel(x, Wq, K_ext, V_ext, Wo):
    my = lax.axis_index("i")
    x2 = x.reshape(SQ, D_MODEL).astype(jnp.bfloat16)
    k2 = K_ext.reshape(SQ, HEAD_COLS).astype(jnp.bfloat16)
    v2 = V_ext.reshape(SQ, HEAD_COLS).astype(jnp.bfloat16)
    wq_s = lax.dynamic_slice(Wq, (0, my * HEAD_COLS), (D_MODEL, HEAD_COLS)).astype(jnp.bfloat16)
    wo_s = lax.dynamic_slice(Wo, (my * HEAD_COLS, 0), (HEAD_COLS, D_MODEL)).astype(jnp.bfloat16)

    out = pl.pallas_call(
        _body,
        out_shape=jax.ShapeDtypeStruct((SQ, D_MODEL), jnp.float32),
        in_specs=[pl.BlockSpec(memory_space=pltpu.VMEM)] * 5,
        out_specs=pl.BlockSpec(memory_space=pltpu.VMEM),
        scratch_shapes=[
            pltpu.VMEM((SQ, HEAD_COLS), jnp.bfloat16),
            pltpu.VMEM((SQ, HEAD_COLS), jnp.bfloat16),
            pltpu.VMEM((SQ, D_MODEL), jnp.float32),
            pltpu.VMEM((512, D_MODEL), jnp.bfloat16),
            pltpu.VMEM((960, D_MODEL), jnp.bfloat16),
            pltpu.VMEM((SQ, D_MODEL), jnp.bfloat16),
            pltpu.VMEM((SQ, SQ), jnp.float32),
            pltpu.SemaphoreType.DMA,
            pltpu.SemaphoreType.DMA((4,)),
            pltpu.SemaphoreType.DMA((4,)),
        ],
        compiler_params=pltpu.CompilerParams(collective_id=0),
    )(x2, wq_s, k2, v2, wo_s)
    return out.reshape(1, SQ, D_MODEL)


# device time: 72407 ns/iter; 2.4423x vs baseline; 1.2438x over previous
import jax
import jax.numpy as jnp
from jax import lax
from jax.experimental import pallas as pl
from jax.experimental.pallas import tpu as pltpu

N_DEV = 16
SQ = 1024
D_MODEL = 1024
HQ_PER = 8
DH = 128
HEAD_COLS = HQ_PER * DH
CHUNK = SQ // N_DEV
HALFC = D_MODEL // 2
SCALE = 0.08838834764831843
BLK = 64

MASKS_A = (1, 3, 4, 8)
MASKS_B = (3, 1, 8, 4)
RS_SIZES = (512, 256, 128, 64)
RS_OFF = (0, 512, 768, 896)


def _bit(m, i):
    return (m >> i) & 1


def _pos_a(c):
    return 8 * (_bit(c, 0) ^ _bit(c, 1)) + 4 * _bit(c, 1) \
        + 2 * _bit(c, 2) + _bit(c, 3)


def _pos_b(c):
    return 8 * _bit(c, 1) + 4 * (_bit(c, 0) ^ _bit(c, 1)) \
        + 2 * _bit(c, 3) + _bit(c, 2)


INV_A = tuple({_pos_a(c): c for c in range(N_DEV)}[q] for q in range(N_DEV))
INV_B = tuple({_pos_b(c): c for c in range(N_DEV)}[q] for q in range(N_DEV))
POS_A = tuple(_pos_a(c) for c in range(N_DEV))
POS_B = tuple(_pos_b(c) for c in range(N_DEV))


def _body(x_ref, wq_ref, k_ref, v_ref, wo_ref, out_ref,
          q_ref, ctx_ref, acc_ref, send_ref, rs_recv_ref, ag_ref,
          bias_ref, send_sem_a, send_sem_b, rs_sems_a, rs_sems_b,
          ag_sems_a, ag_sems_b):
    my = lax.axis_index("i")

    barrier_sem = pltpu.get_barrier_semaphore()
    for msk in (1, 3, 4, 8):
        pl.semaphore_signal(barrier_sem, inc=1, device_id=(my ^ msk,),
                            device_id_type=pl.DeviceIdType.MESH)
    pl.semaphore_wait(barrier_sem, 4)

    q_ref[...] = jnp.dot(x_ref[...], wq_ref[...],
                         preferred_element_type=jnp.float32).astype(
        jnp.bfloat16)

    qb = lax.broadcasted_iota(jnp.int32, (SQ, SQ), 0) // BLK
    kb = lax.broadcasted_iota(jnp.int32, (SQ, SQ), 1) // BLK
    bias_ref[...] = jnp.where(kb <= qb, 0.0, -30000.0).astype(jnp.float32)

    HALF = SQ // 2
    for h in range(HQ_PER):
        sl = slice(h * DH, (h + 1) * DH)
        for (r0, r1, ncols) in ((0, HALF, HALF), (HALF, SQ, SQ)):
            qh = q_ref[r0:r1, sl]
            kh = k_ref[0:ncols, sl]
            s = lax.dot_general(qh, kh, (((1,), (1,)), ((), ())),
                                preferred_element_type=jnp.float32) * SCALE
            s = s + bias_ref[r0:r1, 0:ncols]
            w = jnp.exp(s)
            w = (w / jnp.sum(w, axis=1, keepdims=True)).astype(jnp.bfloat16)
            ctx_ref[r0:r1, sl] = jnp.dot(
                w, v_ref[0:ncols, sl],
                preferred_element_type=jnp.float32).astype(jnp.bfloat16)

    val = jnp.dot(ctx_ref[...], wo_ref[...],
                  preferred_element_type=jnp.float32)
    for q in range(N_DEV):
        ca, cb = INV_A[q], INV_B[q]
        acc_ref[q * CHUNK:(q + 1) * CHUNK, 0:HALFC] = (
            val[ca * CHUNK:(ca + 1) * CHUNK, 0:HALFC])
        acc_ref[q * CHUNK:(q + 1) * CHUNK, HALFC:D_MODEL] = (
            val[cb * CHUNK:(cb + 1) * CHUNK, HALFC:D_MODEL])

    b0 = my & 1
    b1 = (my >> 1) & 1
    b2 = (my >> 2) & 1
    b3 = (my >> 3) & 1
    betas_a = (b0 ^ b1, b1, b2, b3)
    betas_b = (b1, b0 ^ b1, b3, b2)

    lo_a = my * 0
    lo_b = my * 0
    for k in range(4):
        s = RS_SIZES[k]
        send_lo_a = pl.multiple_of(lo_a + (1 - betas_a[k]) * s, 64)
        keep_lo_a = pl.multiple_of(lo_a + betas_a[k] * s, 64)
        send_lo_b = pl.multiple_of(lo_b + (1 - betas_b[k]) * s, 64)
        keep_lo_b = pl.multiple_of(lo_b + betas_b[k] * s, 64)
        send_ref[0:s, 0:HALFC] = (
            acc_ref[pl.ds(send_lo_a, s), 0:HALFC].astype(jnp.bfloat16))
        send_ref[0:s, HALFC:D_MODEL] = (
            acc_ref[pl.ds(send_lo_b, s), HALFC:D_MODEL].astype(jnp.bfloat16))
        rdma_a = pltpu.make_async_remote_copy(
            src_ref=send_ref.at[0:s, 0:HALFC],
            dst_ref=rs_recv_ref.at[RS_OFF[k]:RS_OFF[k] + s, 0:HALFC],
            send_sem=send_sem_a,
            recv_sem=rs_sems_a.at[k],
            device_id=(my ^ MASKS_A[k],),
            device_id_type=pl.DeviceIdType.MESH,
        )
        rdma_b = pltpu.make_async_remote_copy(
            src_ref=send_ref.at[0:s, HALFC:D_MODEL],
            dst_ref=rs_recv_ref.at[RS_OFF[k]:RS_OFF[k] + s, HALFC:D_MODEL],
            send_sem=send_sem_b,
            recv_sem=rs_sems_b.at[k],
            device_id=(my ^ MASKS_B[k],),
            device_id_type=pl.DeviceIdType.MESH,
        )
        rdma_a.start()
        rdma_b.start()
        rdma_a.wait()
        acc_ref[pl.ds(keep_lo_a, s), 0:HALFC] = (
            acc_ref[pl.ds(keep_lo_a, s), 0:HALFC]
            + rs_recv_ref[RS_OFF[k]:RS_OFF[k] + s, 0:HALFC].astype(
                jnp.float32))
        rdma_b.wait()
        acc_ref[pl.ds(keep_lo_b, s), HALFC:D_MODEL] = (
            acc_ref[pl.ds(keep_lo_b, s), HALFC:D_MODEL]
            + rs_recv_ref[RS_OFF[k]:RS_OFF[k] + s, HALFC:D_MODEL].astype(
                jnp.float32))
        lo_a = keep_lo_a
        lo_b = keep_lo_b

    ag_ref[pl.ds(pl.multiple_of(lo_a, 64), CHUNK), 0:HALFC] = (
        acc_ref[pl.ds(pl.multiple_of(lo_a, 64), CHUNK), 0:HALFC].astype(
            jnp.bfloat16))
    ag_ref[pl.ds(pl.multiple_of(lo_b, 64), CHUNK), HALFC:D_MODEL] = (
        acc_ref[pl.ds(pl.multiple_of(lo_b, 64), CHUNK),
                HALFC:D_MODEL].astype(jnp.bfloat16))

    for k in reversed(range(4)):
        sz = CHUNK << (3 - k)
        sl_a = pl.ds(pl.multiple_of(lo_a, 64), sz)
        sl_b = pl.ds(pl.multiple_of(lo_b, 64), sz)
        rdma_a = pltpu.make_async_remote_copy(
            src_ref=ag_ref.at[sl_a, 0:HALFC],
            dst_ref=ag_ref.at[sl_a, 0:HALFC],
            send_sem=send_sem_a,
            recv_sem=ag_sems_a.at[k],
            device_id=(my ^ MASKS_A[k],),
            device_id_type=pl.DeviceIdType.MESH,
        )
        rdma_b = pltpu.make_async_remote_copy(
            src_ref=ag_ref.at[sl_b, HALFC:D_MODEL],
            dst_ref=ag_ref.at[sl_b, HALFC:D_MODEL],
            send_sem=send_sem_b,
            recv_sem=ag_sems_b.at[k],
            device_id=(my ^ MASKS_B[k],),
            device_id_type=pl.DeviceIdType.MESH,
        )
        rdma_a.start()
        rdma_b.start()
        rdma_a.wait()
        rdma_b.wait()
        lo_a = lo_a - (lo_a & sz)
        lo_b = lo_b - (lo_b & sz)

    for b in range(N_DEV):
        pa, pb = POS_A[b], POS_B[b]
        out_ref[b * CHUNK:(b + 1) * CHUNK, 0:HALFC] = (
            ag_ref[pa * CHUNK:(pa + 1) * CHUNK, 0:HALFC].astype(jnp.float32))
        out_ref[b * CHUNK:(b + 1) * CHUNK, HALFC:D_MODEL] = (
            ag_ref[pb * CHUNK:(pb + 1) * CHUNK, HALFC:D_MODEL].astype(
                jnp.float32))


def kernel(x, Wq, K_ext, V_ext, Wo):
    my = lax.axis_index("i")
    x2 = x.reshape(SQ, D_MODEL).astype(jnp.bfloat16)
    k2 = K_ext.reshape(SQ, HEAD_COLS).astype(jnp.bfloat16)
    v2 = V_ext.reshape(SQ, HEAD_COLS).astype(jnp.bfloat16)
    wq_s = lax.dynamic_slice(
        Wq, (0, my * HEAD_COLS), (D_MODEL, HEAD_COLS)).astype(jnp.bfloat16)
    wo_s = lax.dynamic_slice(
        Wo, (my * HEAD_COLS, 0), (HEAD_COLS, D_MODEL)).astype(jnp.bfloat16)

    out = pl.pallas_call(
        _body,
        out_shape=jax.ShapeDtypeStruct((SQ, D_MODEL), jnp.float32),
        in_specs=[pl.BlockSpec(memory_space=pltpu.VMEM)] * 5,
        out_specs=pl.BlockSpec(memory_space=pltpu.VMEM),
        scratch_shapes=[
            pltpu.VMEM((SQ, HEAD_COLS), jnp.bfloat16),
            pltpu.VMEM((SQ, HEAD_COLS), jnp.bfloat16),
            pltpu.VMEM((SQ, D_MODEL), jnp.float32),
            pltpu.VMEM((512, D_MODEL), jnp.bfloat16),
            pltpu.VMEM((960, D_MODEL), jnp.bfloat16),
            pltpu.VMEM((SQ, D_MODEL), jnp.bfloat16),
            pltpu.VMEM((SQ, SQ), jnp.float32),
            pltpu.SemaphoreType.DMA,
            pltpu.SemaphoreType.DMA,
            pltpu.SemaphoreType.DMA((4,)),
            pltpu.SemaphoreType.DMA((4,)),
            pltpu.SemaphoreType.DMA((4,)),
            pltpu.SemaphoreType.DMA((4,)),
        ],
        compiler_params=pltpu.CompilerParams(collective_id=0),
    )(x2, wq_s, k2, v2, wo_s)
    return out.reshape(1, SQ, D_MODEL)


# device time: 66536 ns/iter; 2.6578x vs baseline; 1.0882x over previous
import jax
import jax.numpy as jnp
from jax import lax
from jax.experimental import pallas as pl
from jax.experimental.pallas import tpu as pltpu

N_DEV = 16
SQ = 1024
D_MODEL = 1024
HQ_PER = 8
DH = 128
HEAD_COLS = HQ_PER * DH
CHUNK = SQ // N_DEV
SCALE = 0.08838834764831843
BLK = 64

SLAB_MASKS = ((1, 3, 4, 8), (3, 1, 8, 4), (4, 8, 1, 3))
SLAB_COLS = ((0, 384), (384, 768), (768, 1024))
N_SLAB = len(SLAB_MASKS)
RS_SIZES = (512, 256, 128, 64)
RS_OFF = (0, 512, 768, 896)


def _bit(m, i):
    return (m >> i) & 1


_FUNC = {
    1: lambda c: _bit(c, 0) ^ _bit(c, 1),
    3: lambda c: _bit(c, 1),
    4: lambda c: _bit(c, 2),
    8: lambda c: _bit(c, 3),
}


def _pos(c, masks):
    return (8 * _FUNC[masks[0]](c) + 4 * _FUNC[masks[1]](c)
            + 2 * _FUNC[masks[2]](c) + _FUNC[masks[3]](c))


POS = tuple(tuple(_pos(c, mk) for c in range(N_DEV)) for mk in SLAB_MASKS)
INV = tuple(
    tuple({_pos(c, mk): c for c in range(N_DEV)}[q] for q in range(N_DEV))
    for mk in SLAB_MASKS)


def _body(x_ref, wq_ref, k_ref, v_ref, wo_ref, out_ref,
          q_ref, ctx_ref, acc_ref, send_ref, rs_recv_ref, ag_ref,
          bias_ref, send_sems, rs_sems, ag_sems):
    my = lax.axis_index("i")

    barrier_sem = pltpu.get_barrier_semaphore()
    for msk in (1, 3, 4, 8):
        pl.semaphore_signal(barrier_sem, inc=1, device_id=(my ^ msk,),
                            device_id_type=pl.DeviceIdType.MESH)
    pl.semaphore_wait(barrier_sem, 4)

    q_ref[...] = jnp.dot(x_ref[...], wq_ref[...],
                         preferred_element_type=jnp.float32).astype(
        jnp.bfloat16)

    qb = lax.broadcasted_iota(jnp.int32, (SQ, SQ), 0) // BLK
    kb = lax.broadcasted_iota(jnp.int32, (SQ, SQ), 1) // BLK
    bias_ref[...] = jnp.where(kb <= qb, 0.0, -30000.0).astype(jnp.float32)

    HALF = SQ // 2
    for h in range(HQ_PER):
        sl = slice(h * DH, (h + 1) * DH)
        for (r0, r1, ncols) in ((0, HALF, HALF), (HALF, SQ, SQ)):
            qh = q_ref[r0:r1, sl]
            kh = k_ref[0:ncols, sl]
            s = lax.dot_general(qh, kh, (((1,), (1,)), ((), ())),
                                preferred_element_type=jnp.float32) * SCALE
            w = jnp.exp(s + bias_ref[r0:r1, 0:ncols])
            recip = 1.0 / jnp.sum(w, axis=1, keepdims=True)
            u = jnp.dot(w.astype(jnp.bfloat16), v_ref[0:ncols, sl],
                        preferred_element_type=jnp.float32)
            ctx_ref[r0:r1, sl] = (u * recip).astype(jnp.bfloat16)

    val = jnp.dot(ctx_ref[...], wo_ref[...],
                  preferred_element_type=jnp.float32)
    for q in range(N_DEV):
        for i in range(N_SLAB):
            c0, c1 = SLAB_COLS[i]
            cq = INV[i][q]
            acc_ref[q * CHUNK:(q + 1) * CHUNK, c0:c1] = (
                val[cq * CHUNK:(cq + 1) * CHUNK, c0:c1])

    b0 = my & 1
    b1 = (my >> 1) & 1
    func = {1: b0 ^ b1, 3: b1, 4: (my >> 2) & 1, 8: (my >> 3) & 1}

    lo = [my * 0 for _ in range(N_SLAB)]
    for k in range(4):
        s = RS_SIZES[k]
        rdmas = []
        keep_los = []
        for i in range(N_SLAB):
            c0, c1 = SLAB_COLS[i]
            beta = func[SLAB_MASKS[i][k]]
            send_lo = pl.multiple_of(lo[i] + (1 - beta) * s, 64)
            keep_lo = pl.multiple_of(lo[i] + beta * s, 64)
            keep_los.append(keep_lo)
            send_ref[0:s, c0:c1] = (
                acc_ref[pl.ds(send_lo, s), c0:c1].astype(jnp.bfloat16))
            rdma = pltpu.make_async_remote_copy(
                src_ref=send_ref.at[0:s, c0:c1],
                dst_ref=rs_recv_ref.at[RS_OFF[k]:RS_OFF[k] + s, c0:c1],
                send_sem=send_sems.at[i],
                recv_sem=rs_sems.at[i, k],
                device_id=(my ^ SLAB_MASKS[i][k],),
                device_id_type=pl.DeviceIdType.MESH,
            )
            rdma.start()
            rdmas.append(rdma)
        for i in range(N_SLAB):
            c0, c1 = SLAB_COLS[i]
            rdmas[i].wait()
            acc_ref[pl.ds(keep_los[i], s), c0:c1] = (
                acc_ref[pl.ds(keep_los[i], s), c0:c1]
                + rs_recv_ref[RS_OFF[k]:RS_OFF[k] + s, c0:c1].astype(
                    jnp.float32))
            lo[i] = keep_los[i]

    for i in range(N_SLAB):
        c0, c1 = SLAB_COLS[i]
        sl_i = pl.ds(pl.multiple_of(lo[i], 64), CHUNK)
        ag_ref[sl_i, c0:c1] = acc_ref[sl_i, c0:c1].astype(jnp.bfloat16)

    for k in reversed(range(4)):
        sz = CHUNK << (3 - k)
        rdmas = []
        for i in range(N_SLAB):
            c0, c1 = SLAB_COLS[i]
            sl_i = pl.ds(pl.multiple_of(lo[i], 64), sz)
            rdma = pltpu.make_async_remote_copy(
                src_ref=ag_ref.at[sl_i, c0:c1],
                dst_ref=ag_ref.at[sl_i, c0:c1],
                send_sem=send_sems.at[i],
                recv_sem=ag_sems.at[i, k],
                device_id=(my ^ SLAB_MASKS[i][k],),
                device_id_type=pl.DeviceIdType.MESH,
            )
            rdma.start()
            rdmas.append(rdma)
        for i in range(N_SLAB):
            rdmas[i].wait()
            lo[i] = lo[i] - (lo[i] & sz)

    for b in range(N_DEV):
        for i in range(N_SLAB):
            c0, c1 = SLAB_COLS[i]
            pb = POS[i][b]
            out_ref[b * CHUNK:(b + 1) * CHUNK, c0:c1] = (
                ag_ref[pb * CHUNK:(pb + 1) * CHUNK, c0:c1].astype(
                    jnp.float32))


def kernel(x, Wq, K_ext, V_ext, Wo):
    my = lax.axis_index("i")
    x2 = x.reshape(SQ, D_MODEL).astype(jnp.bfloat16)
    k2 = K_ext.reshape(SQ, HEAD_COLS).astype(jnp.bfloat16)
    v2 = V_ext.reshape(SQ, HEAD_COLS).astype(jnp.bfloat16)
    wq_s = lax.dynamic_slice(
        Wq, (0, my * HEAD_COLS), (D_MODEL, HEAD_COLS)).astype(jnp.bfloat16)
    wo_s = lax.dynamic_slice(
        Wo, (my * HEAD_COLS, 0), (HEAD_COLS, D_MODEL)).astype(jnp.bfloat16)

    out = pl.pallas_call(
        _body,
        out_shape=jax.ShapeDtypeStruct((SQ, D_MODEL), jnp.float32),
        in_specs=[pl.BlockSpec(memory_space=pltpu.VMEM)] * 5,
        out_specs=pl.BlockSpec(memory_space=pltpu.VMEM),
        scratch_shapes=[
            pltpu.VMEM((SQ, HEAD_COLS), jnp.bfloat16),
            pltpu.VMEM((SQ, HEAD_COLS), jnp.bfloat16),
            pltpu.VMEM((SQ, D_MODEL), jnp.float32),
            pltpu.VMEM((512, D_MODEL), jnp.bfloat16),
            pltpu.VMEM((960, D_MODEL), jnp.bfloat16),
            pltpu.VMEM((SQ, D_MODEL), jnp.bfloat16),
            pltpu.VMEM((SQ, SQ), jnp.float32),
            pltpu.SemaphoreType.DMA((N_SLAB,)),
            pltpu.SemaphoreType.DMA((N_SLAB, 4)),
            pltpu.SemaphoreType.DMA((N_SLAB, 4)),
        ],
        compiler_params=pltpu.CompilerParams(collective_id=0),
    )(x2, wq_s, k2, v2, wo_s)
    return out.reshape(1, SQ, D_MODEL)


# device time: 65161 ns/iter; 2.7138x vs baseline; 1.0211x over previous
import jax
import jax.numpy as jnp
from jax import lax
from jax.experimental import pallas as pl
from jax.experimental.pallas import tpu as pltpu

N_DEV = 16
SQ = 1024
D_MODEL = 1024
HQ_PER = 8
DH = 128
HEAD_COLS = HQ_PER * DH
CHUNK = SQ // N_DEV
SCALE = 0.08838834764831843
BLK = 64

SLAB_MASKS = ((1, 3, 4, 8), (3, 1, 8, 4), (4, 8, 1, 3))
SLAB_COLS = ((0, 384), (384, 768), (768, 1024))
N_SLAB = len(SLAB_MASKS)
RS_SIZES = (512, 256, 128, 64)
RS_OFF = (0, 512, 768, 896)


def _bit(m, i):
    return (m >> i) & 1


_FUNC = {
    1: lambda c: _bit(c, 0) ^ _bit(c, 1),
    3: lambda c: _bit(c, 1),
    4: lambda c: _bit(c, 2),
    8: lambda c: _bit(c, 3),
}


def _pos(c, masks):
    return (8 * _FUNC[masks[0]](c) + 4 * _FUNC[masks[1]](c)
            + 2 * _FUNC[masks[2]](c) + _FUNC[masks[3]](c))


POS = tuple(tuple(_pos(c, mk) for c in range(N_DEV)) for mk in SLAB_MASKS)
INV = tuple(
    tuple({_pos(c, mk): c for c in range(N_DEV)}[q] for q in range(N_DEV))
    for mk in SLAB_MASKS)


def _body(x_ref, wq_ref, k_ref, v_ref, wo_ref, out_ref,
          q_ref, ctx_ref, acc_ref, send_ref, rs_recv_ref, ag_ref,
          bias_ref, send_sems, rs_sems, ag_sems):
    my = lax.axis_index("i")

    barrier_sem = pltpu.get_barrier_semaphore()
    for msk in (1, 3, 4, 8):
        pl.semaphore_signal(barrier_sem, inc=1, device_id=(my ^ msk,),
                            device_id_type=pl.DeviceIdType.MESH)
    pl.semaphore_wait(barrier_sem, 4)

    q_ref[...] = jnp.dot(x_ref[...], wq_ref[...],
                         preferred_element_type=jnp.float32).astype(
        jnp.bfloat16)

    qb = lax.broadcasted_iota(jnp.int32, (SQ, SQ), 0) // BLK
    kb = lax.broadcasted_iota(jnp.int32, (SQ, SQ), 1) // BLK
    bias_ref[...] = jnp.where(kb <= qb, 0.0, -30000.0).astype(jnp.float32)

    HALF = SQ // 2
    for h in range(HQ_PER):
        sl = slice(h * DH, (h + 1) * DH)
        for (r0, r1, ncols) in ((0, HALF, HALF), (HALF, SQ, SQ)):
            qh = q_ref[r0:r1, sl]
            kh = k_ref[0:ncols, sl]
            s = lax.dot_general(qh, kh, (((1,), (1,)), ((), ())),
                                preferred_element_type=jnp.float32) * SCALE
            w = jnp.exp(s + bias_ref[r0:r1, 0:ncols])
            recip = 1.0 / jnp.sum(w, axis=1, keepdims=True)
            u = jnp.dot(w.astype(jnp.bfloat16), v_ref[0:ncols, sl],
                        preferred_element_type=jnp.float32)
            ctx_ref[r0:r1, sl] = (u * recip).astype(jnp.bfloat16)

    val = jnp.dot(ctx_ref[...], wo_ref[...],
                  preferred_element_type=jnp.float32)
    for q in range(N_DEV):
        for i in range(N_SLAB):
            c0, c1 = SLAB_COLS[i]
            cq = INV[i][q]
            acc_ref[q * CHUNK:(q + 1) * CHUNK, c0:c1] = (
                val[cq * CHUNK:(cq + 1) * CHUNK, c0:c1])

    b0 = my & 1
    b1 = (my >> 1) & 1
    func = {1: b0 ^ b1, 3: b1, 4: (my >> 2) & 1, 8: (my >> 3) & 1}

    lo = [my * 0 for _ in range(N_SLAB)]
    for k in range(3):
        s = RS_SIZES[k]
        rdmas = []
        keep_los = []
        for i in range(N_SLAB):
            c0, c1 = SLAB_COLS[i]
            beta = func[SLAB_MASKS[i][k]]
            send_lo = pl.multiple_of(lo[i] + (1 - beta) * s, 64)
            keep_lo = pl.multiple_of(lo[i] + beta * s, 64)
            keep_los.append(keep_lo)
            send_ref[0:s, c0:c1] = (
                acc_ref[pl.ds(send_lo, s), c0:c1].astype(jnp.bfloat16))
            rdma = pltpu.make_async_remote_copy(
                src_ref=send_ref.at[0:s, c0:c1],
                dst_ref=rs_recv_ref.at[RS_OFF[k]:RS_OFF[k] + s, c0:c1],
                send_sem=send_sems.at[i],
                recv_sem=rs_sems.at[i, k],
                device_id=(my ^ SLAB_MASKS[i][k],),
                device_id_type=pl.DeviceIdType.MESH,
            )
            rdma.start()
            rdmas.append(rdma)
        for i in range(N_SLAB):
            c0, c1 = SLAB_COLS[i]
            rdmas[i].wait()
            acc_ref[pl.ds(keep_los[i], s), c0:c1] = (
                acc_ref[pl.ds(keep_los[i], s), c0:c1]
                + rs_recv_ref[RS_OFF[k]:RS_OFF[k] + s, c0:c1].astype(
                    jnp.float32))
            lo[i] = keep_los[i]

    rdmas = []
    sls = []
    for i in range(N_SLAB):
        c0, c1 = SLAB_COLS[i]
        sl_i = pl.ds(pl.multiple_of(lo[i], 128), 128)
        sls.append(sl_i)
        send_ref[0:128, c0:c1] = acc_ref[sl_i, c0:c1].astype(jnp.bfloat16)
        rdma = pltpu.make_async_remote_copy(
            src_ref=send_ref.at[0:128, c0:c1],
            dst_ref=rs_recv_ref.at[896:1024, c0:c1],
            send_sem=send_sems.at[i],
            recv_sem=rs_sems.at[i, 3],
            device_id=(my ^ SLAB_MASKS[i][3],),
            device_id_type=pl.DeviceIdType.MESH,
        )
        rdma.start()
        rdmas.append(rdma)
    for i in range(N_SLAB):
        c0, c1 = SLAB_COLS[i]
        rdmas[i].wait()
        ag_ref[sls[i], c0:c1] = (
            acc_ref[sls[i], c0:c1]
            + rs_recv_ref[896:1024, c0:c1].astype(jnp.float32)
        ).astype(jnp.bfloat16)

    for k in reversed(range(3)):
        sz = 128 << (2 - k)
        rdmas = []
        for i in range(N_SLAB):
            c0, c1 = SLAB_COLS[i]
            sl_i = pl.ds(pl.multiple_of(lo[i], 64), sz)
            rdma = pltpu.make_async_remote_copy(
                src_ref=ag_ref.at[sl_i, c0:c1],
                dst_ref=ag_ref.at[sl_i, c0:c1],
                send_sem=send_sems.at[i],
                recv_sem=ag_sems.at[i, k],
                device_id=(my ^ SLAB_MASKS[i][k],),
                device_id_type=pl.DeviceIdType.MESH,
            )
            rdma.start()
            rdmas.append(rdma)
        for i in range(N_SLAB):
            rdmas[i].wait()
            lo[i] = lo[i] - (lo[i] & sz)

    for b in range(N_DEV):
        for i in range(N_SLAB):
            c0, c1 = SLAB_COLS[i]
            pb = POS[i][b]
            out_ref[b * CHUNK:(b + 1) * CHUNK, c0:c1] = (
                ag_ref[pb * CHUNK:(pb + 1) * CHUNK, c0:c1].astype(
                    jnp.float32))


def kernel(x, Wq, K_ext, V_ext, Wo):
    my = lax.axis_index("i")
    x2 = x.reshape(SQ, D_MODEL).astype(jnp.bfloat16)
    k2 = K_ext.reshape(SQ, HEAD_COLS).astype(jnp.bfloat16)
    v2 = V_ext.reshape(SQ, HEAD_COLS).astype(jnp.bfloat16)
    wq_s = lax.dynamic_slice(
        Wq, (0, my * HEAD_COLS), (D_MODEL, HEAD_COLS)).astype(jnp.bfloat16)
    wo_s = lax.dynamic_slice(
        Wo, (my * HEAD_COLS, 0), (HEAD_COLS, D_MODEL)).astype(jnp.bfloat16)

    out = pl.pallas_call(
        _body,
        out_shape=jax.ShapeDtypeStruct((SQ, D_MODEL), jnp.float32),
        in_specs=[pl.BlockSpec(memory_space=pltpu.VMEM)] * 5,
        out_specs=pl.BlockSpec(memory_space=pltpu.VMEM),
        scratch_shapes=[
            pltpu.VMEM((SQ, HEAD_COLS), jnp.bfloat16),
            pltpu.VMEM((SQ, HEAD_COLS), jnp.bfloat16),
            pltpu.VMEM((SQ, D_MODEL), jnp.float32),
            pltpu.VMEM((512, D_MODEL), jnp.bfloat16),
            pltpu.VMEM((1024, D_MODEL), jnp.bfloat16),
            pltpu.VMEM((SQ, D_MODEL), jnp.bfloat16),
            pltpu.VMEM((SQ, SQ), jnp.float32),
            pltpu.SemaphoreType.DMA((N_SLAB,)),
            pltpu.SemaphoreType.DMA((N_SLAB, 4)),
            pltpu.SemaphoreType.DMA((N_SLAB, 4)),
        ],
        compiler_params=pltpu.CompilerParams(collective_id=0),
    )(x2, wq_s, k2, v2, wo_s)
    return out.reshape(1, SQ, D_MODEL)
